# Initial kernel scaffold; baseline (speedup 1.0000x reference)
import jax
import jax.numpy as jnp
from jax.experimental import pallas as pl


def kernel(features, agent_order, pressed_id, W0, a_src0, a_dst0, b0, W1, a_src1, a_dst1, b1, W2, a_src2, a_dst2, b2, Wc, bc):
    raise NotImplementedError("write your pallas kernel here")


# speedup vs baseline: 298.4045x; 298.4045x over previous
import numpy as np
import jax
import jax.numpy as jnp
from jax.experimental import pallas as pl

B, T, A, F_IN = 32, 100, 23, 16
HID = 64
HEADS = 4
NF = B * T
N = NF * A

FB = 16
FBA = FB * A
GRID = NF // FB

_row = np.arange(FBA)
_frame = _row // A
_valid = (_frame[:, None] == _frame[None, :]) & (_row[:, None] != _row[None, :])
_MASK = np.where(_valid, 0.0, -1e9).astype(np.float32)
_POOL = (np.repeat(np.eye(FB, dtype=np.float32), A, axis=1) / A)

_DN_T = (((1,), (1,)), ((), ()))


def _head_rows(a_vec):
    eye = jnp.eye(HEADS, dtype=a_vec.dtype)
    return (eye[:, :, None] * a_vec[None, :, :]).reshape(HEADS, HEADS * HID)


def _gat(x, W, asT, adT, b, mask, heads, hid):
    xw = jnp.dot(x, W, preferred_element_type=jnp.float32)
    s_src = jax.lax.dot_general(asT, xw, _DN_T,
                                preferred_element_type=jnp.float32)
    s_dst = jax.lax.dot_general(xw, adT, _DN_T,
                                preferred_element_type=jnp.float32)
    outs = []
    for h in range(heads):
        e = s_dst[:, h:h + 1] + s_src[h:h + 1, :]
        e = jnp.maximum(e, 0.2 * e) + mask
        m = jnp.max(e, axis=1, keepdims=True)
        ex = jnp.exp(e - m)
        den = jnp.sum(ex, axis=1, keepdims=True) + 1e-16
        oh = jnp.dot(ex, xw[:, h * hid:(h + 1) * hid],
                     preferred_element_type=jnp.float32)
        outs.append(oh / den)
    out = jnp.concatenate(outs, axis=1) if heads > 1 else outs[0]
    return out + b


def _body(x_ref, w0, ast0, adt0, b0, w1, ast1, adt1, b1,
          w2, ast2, adt2, b2, wcT, bc, mask_ref, pool_ref, out_ref):
    mask = mask_ref[...]
    x = jax.nn.relu(_gat(x_ref[...], w0[...], ast0[...], adt0[...], b0[...],
                         mask, HEADS, HID))
    x = jax.nn.relu(_gat(x, w1[...], ast1[...], adt1[...], b1[...],
                         mask, HEADS, HID))
    x = jax.nn.relu(_gat(x, w2[...], ast2[...], adt2[...], b2[...],
                         mask, 1, HID))
    pooled = jnp.dot(pool_ref[...], x, preferred_element_type=jnp.float32)
    logits = jax.lax.dot_general(wcT[...], pooled, _DN_T,
                                 preferred_element_type=jnp.float32)
    out_ref[0] = logits + bc[...]


def kernel(features, agent_order, pressed_id, W0, a_src0, a_dst0, b0,
           W1, a_src1, a_dst1, b1, W2, a_src2, a_dst2, b2, Wc, bc):
    x = features.reshape(N, F_IN)
    ast0, adt0 = _head_rows(a_src0), _head_rows(a_dst0)
    ast1, adt1 = _head_rows(a_src1), _head_rows(a_dst1)
    ast2, adt2 = a_src2, a_dst2
    b0r, b1r, b2r = b0.reshape(1, -1), b1.reshape(1, -1), b2.reshape(1, -1)
    wcT = Wc.reshape(1, HID)
    bcr = jnp.broadcast_to(bc.reshape(1, 1), (1, FB))
    mask = jnp.asarray(_MASK)
    pool = jnp.asarray(_POOL)

    const = lambda i: (0, 0)
    C = HEADS * HID
    out = pl.pallas_call(
        _body,
        grid=(GRID,),
        in_specs=[
            pl.BlockSpec((FBA, F_IN), lambda i: (i, 0)),
            pl.BlockSpec((F_IN, C), const),
            pl.BlockSpec((HEADS, C), const),
            pl.BlockSpec((HEADS, C), const),
            pl.BlockSpec((1, C), const),
            pl.BlockSpec((C, C), const),
            pl.BlockSpec((HEADS, C), const),
            pl.BlockSpec((HEADS, C), const),
            pl.BlockSpec((1, C), const),
            pl.BlockSpec((C, HID), const),
            pl.BlockSpec((1, HID), const),
            pl.BlockSpec((1, HID), const),
            pl.BlockSpec((1, HID), const),
            pl.BlockSpec((1, HID), const),
            pl.BlockSpec((1, FB), const),
            pl.BlockSpec((FBA, FBA), const),
            pl.BlockSpec((FB, FBA), const),
        ],
        out_specs=pl.BlockSpec((1, 1, FB), lambda i: (i, 0, 0)),
        out_shape=jax.ShapeDtypeStruct((GRID, 1, FB), jnp.float32),
    )(x, W0, ast0, adt0, b0r, W1, ast1, adt1, b1r,
      W2, ast2, adt2, b2r, wcT, bcr, mask, pool)
    return out.reshape(NF)
